# baseline (speedup 1.0000x reference)
import jax
import jax.numpy as jnp
from jax.experimental import pallas as pl

N = 10000
B = 64
H = 256
NEG_SLOPE = 0.2


def _head_body(graph_emb_ref, glob_ref, fc0_W_ref, fc0_b_ref, fc1_W_ref,
               fc1_b_ref, fc2_W_ref, fc2_b_ref, ln_w_ref, ln_b_ref, q_W_ref,
               q_b_ref, out_ref):
    x = jnp.concatenate([graph_emb_ref[...], glob_ref[...]], axis=1)
    x = jax.nn.relu(x @ fc0_W_ref[...] + fc0_b_ref[...])
    x = jax.nn.relu(x @ fc1_W_ref[...] + fc1_b_ref[...])
    x = jax.nn.relu(x @ fc2_W_ref[...] + fc2_b_ref[...])
    mu = jnp.mean(x, axis=-1, keepdims=True)
    var = jnp.mean((x - mu) ** 2, axis=-1, keepdims=True)
    x = (x - mu) / jnp.sqrt(var + 1e-5) * ln_w_ref[...] + ln_b_ref[...]
    out_ref[...] = x @ q_W_ref[...] + q_b_ref[...]


def _head(graph_emb, glob, fc0_W, fc0_b, fc1_W, fc1_b, fc2_W, fc2_b,
          ln_w, ln_b, q_W, q_b):
    return pl.pallas_call(
        _head_body,
        out_shape=jax.ShapeDtypeStruct((B, 1), jnp.float32),
    )(graph_emb, glob, fc0_W, fc0_b.reshape(1, H), fc1_W, fc1_b.reshape(1, H),
      fc2_W, fc2_b.reshape(1, H), ln_w.reshape(1, H), ln_b.reshape(1, H),
      q_W, q_b.reshape(1, 1))


def _gatv2(x, src, dst, Wl, bl, Wr, br, att, bias, n):
    xl = x @ Wl + bl
    xr = x @ Wr + br
    m = jax.nn.leaky_relu(xl[src] + xr[dst], NEG_SLOPE)
    e = jnp.sum(m * att, axis=-1)
    emax = jax.ops.segment_max(e, dst, num_segments=n)
    ez = jnp.exp(e - emax[dst])
    esum = jax.ops.segment_sum(ez, dst, num_segments=n)
    alpha = ez / (esum[dst] + 1e-16)
    out = jax.ops.segment_sum(xl[src] * alpha[:, None], dst, num_segments=n)
    return out + bias


def kernel(node_features, edge_index, batch_id, is_final, emb_W, emb_b,
           g0_Wl, g0_bl, g0_Wr, g0_br, g0_att, g0_bias,
           g1_Wl, g1_bl, g1_Wr, g1_br, g1_att, g1_bias,
           glob_W, glob_b, fc0_W, fc0_b, fc1_W, fc1_b, fc2_W, fc2_b,
           ln_w, ln_b, q_W, q_b):
    n = node_features.shape[0]
    loop = jnp.arange(n, dtype=edge_index.dtype)
    src = jnp.concatenate([edge_index[0], loop])
    dst = jnp.concatenate([edge_index[1], loop])
    x = node_features @ emb_W + emb_b
    x = jax.nn.relu(_gatv2(x, src, dst, g0_Wl, g0_bl, g0_Wr, g0_br, g0_att, g0_bias, n))
    x = jax.nn.relu(_gatv2(x, src, dst, g1_Wl, g1_bl, g1_Wr, g1_br, g1_att, g1_bias, n))
    gsum = jax.ops.segment_sum(x, batch_id, num_segments=B)
    cnt = jax.ops.segment_sum(jnp.ones((n,), jnp.float32), batch_id, num_segments=B)
    graph_emb = gsum / jnp.maximum(cnt, 1.0)[:, None]
    glob = is_final @ glob_W + glob_b
    return _head(graph_emb, glob, fc0_W, fc0_b, fc1_W, fc1_b, fc2_W, fc2_b,
                 ln_w, ln_b, q_W, q_b)


# speedup vs baseline: 1.1857x; 1.1857x over previous
import functools

import jax
import jax.numpy as jnp
from jax import lax
from jax.experimental import pallas as pl
from jax.experimental.pallas import tpu as pltpu
from jax.experimental.pallas import tpu_sc as plsc

N = 10000
E = 320000
D = 128
H = 256
B = 64
NEG_SLOPE = 0.2

NC = 2
NS = 16
L = 16
HV = H // L
HH = H // 2
HQ = H // 4
HV2 = HH // L
QV = HQ // L

E2 = E + N
E2P = 330240
NW = NC * NS
NBIN = 32
TROW = 320
BCAP = 64
KP1 = 16
KP2 = 64
EP32 = E2P // NW
SEGCAP = 10368


def _emb_body(nf_ref, eW_ref, eb_ref, x0_ref, x1_ref, x2_ref, x3_ref):
    x = jnp.dot(nf_ref[...], eW_ref[...], preferred_element_type=jnp.float32)
    x = x + eb_ref[...]
    x0_ref[...] = x[:, 0 * HQ:1 * HQ]
    x1_ref[...] = x[:, 1 * HQ:2 * HQ]
    x2_ref[...] = x[:, 2 * HQ:3 * HQ]
    x3_ref[...] = x[:, 3 * HQ:4 * HQ]


def _to_halves_body(h0_ref, h1_ref, h2_ref, h3_ref, a_ref, b_ref):
    a_ref[...] = jnp.concatenate([h0_ref[...], h1_ref[...]], axis=1)
    b_ref[...] = jnp.concatenate([h2_ref[...], h3_ref[...]], axis=1)


def _emb(nf, eW, eb):
    R = 1000
    return pl.pallas_call(
        _emb_body,
        grid=(N // R,),
        in_specs=[
            pl.BlockSpec((R, D), lambda i: (i, 0)),
            pl.BlockSpec((D, H), lambda i: (0, 0)),
            pl.BlockSpec((1, H), lambda i: (0, 0)),
        ],
        out_specs=[pl.BlockSpec((R, HQ), lambda i: (i, 0))] * 4,
        out_shape=[jax.ShapeDtypeStruct((N, HQ), jnp.float32)] * 4,
    )(nf, eW, eb.reshape(1, H))


def _mm2_body(h0_ref, h1_ref, h2_ref, h3_ref, Wl_ref, bl_ref, Wr_ref, br_ref,
              xlA_ref, xlB_ref, xrA_ref, xrB_ref):
    x = jnp.concatenate([h0_ref[...], h1_ref[...], h2_ref[...], h3_ref[...]],
                        axis=1)
    xl = jnp.dot(x, Wl_ref[...], preferred_element_type=jnp.float32) + bl_ref[...]
    xr = jnp.dot(x, Wr_ref[...], preferred_element_type=jnp.float32) + br_ref[...]
    xlA_ref[...] = xl[:, :HH]
    xlB_ref[...] = xl[:, HH:]
    xrA_ref[...] = xr[:, :HH]
    xrB_ref[...] = xr[:, HH:]


def _mm2(hq, Wl, bl, Wr, br):
    R = 1000
    full = lambda i: (0, 0)
    return pl.pallas_call(
        _mm2_body,
        grid=(N // R,),
        in_specs=[pl.BlockSpec((R, HQ), lambda i: (i, 0))] * 4 + [
            pl.BlockSpec((H, H), full),
            pl.BlockSpec((1, H), full),
            pl.BlockSpec((H, H), full),
            pl.BlockSpec((1, H), full),
        ],
        out_specs=[pl.BlockSpec((R, HH), lambda i: (i, 0))] * 4,
        out_shape=[jax.ShapeDtypeStruct((N, HH), jnp.float32)] * 4,
    )(*hq, Wl, bl.reshape(1, H), Wr, br.reshape(1, H))


def _head_body(h0_ref, h1_ref, h2_ref, h3_ref, bidf_ref, isf_ref, globW_ref,
               globb_ref, fc0_W_ref, fc0_b_ref, fc1_W_ref, fc1_b_ref,
               fc2_W_ref, fc2_b_ref, ln_w_ref, ln_b_ref, q_W_ref, q_b_ref,
               out_ref):
    bid = bidf_ref[...]
    gid = lax.broadcasted_iota(jnp.int32, (B, N), 0).astype(jnp.float32)
    onehot = jnp.where(jnp.broadcast_to(bid, (B, N)) == gid, 1.0, 0.0)
    h = jnp.concatenate([h0_ref[...], h1_ref[...], h2_ref[...], h3_ref[...]],
                        axis=1)
    gsum = jnp.dot(onehot, h, preferred_element_type=jnp.float32)
    cnt = jnp.sum(onehot, axis=1, keepdims=True)
    graph_emb = gsum / jnp.maximum(cnt, 1.0)
    isf = isf_ref[...]
    glob = (isf[:, 0:1] * globW_ref[0:1, :] + isf[:, 1:2] * globW_ref[1:2, :]
            + globb_ref[...])
    x = jnp.concatenate([graph_emb, glob], axis=1)
    x = jax.nn.relu(jnp.dot(x, fc0_W_ref[...], preferred_element_type=jnp.float32) + fc0_b_ref[...])
    x = jax.nn.relu(jnp.dot(x, fc1_W_ref[...], preferred_element_type=jnp.float32) + fc1_b_ref[...])
    x = jax.nn.relu(jnp.dot(x, fc2_W_ref[...], preferred_element_type=jnp.float32) + fc2_b_ref[...])
    mu = jnp.mean(x, axis=-1, keepdims=True)
    var = jnp.mean((x - mu) ** 2, axis=-1, keepdims=True)
    x = (x - mu) / jnp.sqrt(var + 1e-5) * ln_w_ref[...] + ln_b_ref[...]
    out_ref[...] = jnp.dot(x, q_W_ref[...], preferred_element_type=jnp.float32) + q_b_ref[...]


def _head(hq, batch_id, is_final, glob_W, glob_b, fc0_W, fc0_b,
          fc1_W, fc1_b, fc2_W, fc2_b, ln_w, ln_b, q_W, q_b):
    bidf = batch_id.astype(jnp.float32).reshape(1, N)
    return pl.pallas_call(
        _head_body,
        out_shape=jax.ShapeDtypeStruct((B, 1), jnp.float32),
    )(*hq, bidf, is_final, glob_W, glob_b.reshape(1, H),
      fc0_W, fc0_b.reshape(1, H), fc1_W, fc1_b.reshape(1, H),
      fc2_W, fc2_b.reshape(1, H), ln_w.reshape(1, H), ln_b.reshape(1, H),
      q_W, q_b.reshape(1, 1))


def _p1_body(xlA, xlB, xrA, xrB, edges_hbm, att_hbm,
             pv_seg, w_seg, seg_cnt,
             e_ch, idx_src, idx_dst,
             rA0, rA1, rB0, rB1, att_v,
             bb_pv, bb_w, bincnt, flushed, totc, semA, semB):
    c = lax.axis_index("c")
    s = lax.axis_index("s")
    q = s * NC + c
    rA = (rA0, rA1)
    rB = (rB0, rB1)
    xls = (xlA, xlB)
    xrs = (xrA, xrB)

    pltpu.sync_copy(att_hbm, att_v)

    def _init_bin(b, _):
        bsplat = jnp.full((L,), b, jnp.int32) * TROW
        for g in range(BCAP // L):
            bb_pv[b, pl.ds(g * L, L)] = bsplat
            bb_w[b, pl.ds(g * L, L)] = jnp.zeros((L,), jnp.float32)
        return 0
    lax.fori_loop(0, NBIN, _init_bin, 0)
    bincnt[pl.ds(0, L)] = jnp.zeros((L,), jnp.int32)
    bincnt[pl.ds(L, L)] = jnp.zeros((L,), jnp.int32)
    flushed[pl.ds(0, L)] = jnp.zeros((L,), jnp.int32)
    flushed[pl.ds(L, L)] = jnp.zeros((L,), jnp.int32)

    def _flush(b_s, bin16):
        f16 = plsc.load_gather(flushed, [bin16])
        off = (q * NBIN + b_s) * SEGCAP + f16[0] * BCAP
        pltpu.sync_copy(bb_pv.at[b_s], pv_seg.at[pl.ds(off, BCAP)])
        pltpu.sync_copy(bb_w.at[b_s], w_seg.at[pl.ds(off, BCAP)])
        plsc.store_scatter(flushed, [bin16], f16 + 1)
        plsc.store_scatter(bincnt, [bin16], jnp.zeros((L,), jnp.int32))
        bsplat = jnp.full((L,), b_s, jnp.int32) * TROW
        for g in range(BCAP // L):
            bb_pv[b_s, pl.ds(g * L, L)] = bsplat
            bb_w[b_s, pl.ds(g * L, L)] = jnp.zeros((L,), jnp.float32)

    ebase = q * EP32

    def _chunk(j, _):
        base = ebase + j * KP1
        pltpu.sync_copy(edges_hbm.at[pl.ds(base, KP1)], e_ch)
        pvv = e_ch[pl.ds(0, L)]
        idx_src[pl.ds(0, L)] = pvv >> 14
        idx_dst[pl.ds(0, L)] = jnp.minimum(pvv & (2 ** 14 - 1), N - 1)
        cps = [pltpu.async_copy(xls[k].at[idx_src], rA[k], semA)
               for k in range(2)]
        cps += [pltpu.async_copy(xrs[k].at[idx_dst], rB[k], semB)
                for k in range(2)]
        for cp in cps:
            cp.wait()
        att_regs = [att_v[pl.ds(hh * L, L)] for hh in range(HV)]

        def _edge(e, _):
            acc = jnp.zeros((L,), jnp.float32)
            for hh in range(HV):
                a = rA[hh // HV2][e, pl.ds((hh % HV2) * L, L)]
                b = rB[hh // HV2][e, pl.ds((hh % HV2) * L, L)]
                z = a + b
                zz = jnp.where(z > 0, z, z * NEG_SLOPE)
                acc = acc + zz * att_regs[hh]
            es = jnp.sum(acc)
            wv = jnp.exp(jnp.full((L,), es))
            pvf = plsc.load_gather(e_ch, [jnp.full((L,), e, jnp.int32)])
            dvf = pvf & (2 ** 14 - 1)
            bin16 = (dvf * 6554) >> 21
            cnt16 = plsc.load_gather(bincnt, [bin16])
            plsc.store_scatter(bb_pv, [bin16, cnt16], pvf)
            plsc.store_scatter(bb_w, [bin16, cnt16], wv)
            newc = cnt16 + 1
            plsc.store_scatter(bincnt, [bin16], newc)

            @pl.when(newc[0] == BCAP)
            def _():
                _flush(bin16[0], bin16)
            return 0

        lax.fori_loop(0, KP1, _edge, 0)
        return 0

    lax.fori_loop(0, EP32 // KP1, _chunk, 0)

    for b in range(NBIN):
        bin16 = jnp.full((L,), b, jnp.int32)
        c16 = plsc.load_gather(bincnt, [bin16])

        @pl.when(c16[0] > 0)
        def _():
            _flush(b, bin16)
    totc[pl.ds(0, L)] = flushed[pl.ds(0, L)] * BCAP
    totc[pl.ds(L, L)] = flushed[pl.ds(L, L)] * BCAP
    pltpu.sync_copy(totc, seg_cnt.at[q])
    plsc.subcore_barrier()


def _p2_body(pv_seg, w_seg, seg_cnt, xlA, xlB, bias_hbm,
             h0, h1, h2, h3,
             accum, den, ch_pv, ch_w, lr_buf, idx_src,
             rows, epb, cnts_v, bias_v, semA):
    c = lax.axis_index("c")
    s = lax.axis_index("s")
    t = s * NC + c
    zero16 = jnp.zeros((L,), jnp.float32)
    tbase = t * TROW
    xls = (xlA, xlB)
    hs = (h0, h1, h2, h3)

    pltpu.sync_copy(seg_cnt, cnts_v)
    pltpu.sync_copy(bias_hbm, bias_v)

    for quarter in range(4):
        xl_hbm = xls[quarter // 2]
        coff = (quarter % 2) * HQ
        h_hbm = hs[quarter]

        def _zero_acc(r, _):
            for hh in range(QV):
                accum[r, pl.ds(hh * L, L)] = zero16
            return 0
        lax.fori_loop(0, TROW, _zero_acc, 0)
        if quarter == 0:
            for g in range(TROW // L + 1):
                den[pl.ds(g * L, L)] = zero16

        def _scanner(qq, _):
            cq = plsc.load_gather(cnts_v, [jnp.full((L,), qq, jnp.int32),
                                           jnp.full((L,), t, jnp.int32)])
            nch = cq[0] >> 6

            def _chunk(j, _):
                off = (qq * NBIN + t) * SEGCAP + j * BCAP
                pltpu.sync_copy(pv_seg.at[pl.ds(off, BCAP)], ch_pv)
                pltpu.sync_copy(w_seg.at[pl.ds(off, BCAP)], ch_w)
                for g in range(BCAP // L):
                    pvv = ch_pv[pl.ds(g * L, L)]
                    idx_src[pl.ds(g * L, L)] = pvv >> 14
                    lr_buf[pl.ds(g * L, L)] = (pvv & (2 ** 14 - 1)) - tbase
                pltpu.async_copy(xl_hbm.at[idx_src], rows, semA).wait()

                def _edge(e, _):
                    lr_s = lr_buf[pl.ds(e, L)][0]
                    lr16 = jnp.full((L,), lr_s, jnp.int32)
                    wspl = plsc.load_gather(
                        ch_w, [jnp.full((L,), e, jnp.int32)])
                    for hh in range(QV):
                        cur = accum[lr_s, pl.ds(hh * L, L)]
                        accum[lr_s, pl.ds(hh * L, L)] = (
                            cur + rows[e, pl.ds(coff + hh * L, L)] * wspl)
                    if quarter == 0:
                        curd = plsc.load_gather(den, [lr16])
                        plsc.store_scatter(den, [lr16], curd + wspl)
                    return 0

                lax.fori_loop(0, BCAP, _edge, 0)
                return 0

            lax.fori_loop(0, nch, _chunk, 0)
            return 0

        lax.fori_loop(0, NW, _scanner, 0)

        bias_regs = [bias_v[pl.ds(quarter * HQ + hh * L, L)]
                     for hh in range(QV)]

        def _ep_rows(c0, nr, out_n):
            def _ep_row(r, _):
                d = den[pl.ds(c0 + r, L)][0]
                inv = 1.0 / (jnp.full((L,), d) + 1e-16)
                for hh in range(QV):
                    v = accum[c0 + r, pl.ds(hh * L, L)] * inv + bias_regs[hh]
                    epb[r, pl.ds(hh * L, L)] = jnp.maximum(v, 0.0)
                return 0

            lax.fori_loop(0, nr, _ep_row, 0)
            pltpu.sync_copy(epb.at[pl.ds(0, out_n)],
                            h_hbm.at[pl.ds(tbase + c0, out_n)])

        for cb in range(TROW // KP2):
            @pl.when(tbase + (cb + 1) * KP2 <= N)
            def _():
                _ep_rows(cb * KP2, KP2, KP2)

        @pl.when(t == NW - 1)
        def _():
            _ep_rows(KP2, L, L)
    plsc.subcore_barrier()


def _sc_gat(xlA, xlB, xrA, xrB, edges, att, bias):
    mesh = plsc.VectorSubcoreMesh(core_axis_name="c", subcore_axis_name="s",
                                  num_cores=NC, num_subcores=NS)
    p1 = functools.partial(
        pl.kernel,
        out_type=(
            jax.ShapeDtypeStruct((NW * NBIN * SEGCAP,), jnp.int32),
            jax.ShapeDtypeStruct((NW * NBIN * SEGCAP,), jnp.float32),
            jax.ShapeDtypeStruct((NW, NBIN), jnp.int32),
        ),
        mesh=mesh,
        compiler_params=pltpu.CompilerParams(needs_layout_passes=False),
        scratch_types=[
            pltpu.VMEM((KP1,), jnp.int32),
            pltpu.VMEM((KP1,), jnp.int32),
            pltpu.VMEM((KP1,), jnp.int32),
        ] + [pltpu.VMEM((KP1, HH), jnp.float32)] * 4 + [
            pltpu.VMEM((H,), jnp.float32),
            pltpu.VMEM((NBIN, BCAP), jnp.int32),
            pltpu.VMEM((NBIN, BCAP), jnp.float32),
            pltpu.VMEM((NBIN,), jnp.int32),
            pltpu.VMEM((NBIN,), jnp.int32),
            pltpu.VMEM((NBIN,), jnp.int32),
            pltpu.SemaphoreType.DMA,
            pltpu.SemaphoreType.DMA,
        ],
    )(_p1_body)
    pv_seg, w_seg, seg_cnt = p1(xlA, xlB, xrA, xrB, edges, att)

    p2 = functools.partial(
        pl.kernel,
        out_type=tuple(jax.ShapeDtypeStruct((N, HQ), jnp.float32)
                       for _ in range(4)),
        mesh=mesh,
        compiler_params=pltpu.CompilerParams(needs_layout_passes=False),
        scratch_types=[
            pltpu.VMEM((TROW, HQ), jnp.float32),
            pltpu.VMEM((TROW + L, ), jnp.float32),
            pltpu.VMEM((BCAP,), jnp.int32),
            pltpu.VMEM((BCAP,), jnp.float32),
            pltpu.VMEM((BCAP + L,), jnp.int32),
            pltpu.VMEM((BCAP,), jnp.int32),
            pltpu.VMEM((BCAP, HH), jnp.float32),
            pltpu.VMEM((BCAP, HQ), jnp.float32),
            pltpu.VMEM((NW, NBIN), jnp.int32),
            pltpu.VMEM((H,), jnp.float32),
            pltpu.SemaphoreType.DMA,
        ],
    )(_p2_body)
    return p2(pv_seg, w_seg, seg_cnt, xlA, xlB, bias)


def kernel(node_features, edge_index, batch_id, is_final, emb_W, emb_b,
           g0_Wl, g0_bl, g0_Wr, g0_br, g0_att, g0_bias,
           g1_Wl, g1_bl, g1_Wr, g1_br, g1_att, g1_bias,
           glob_W, glob_b, fc0_W, fc0_b, fc1_W, fc1_b, fc2_W, fc2_b,
           ln_w, ln_b, q_W, q_b):
    loop = jnp.arange(N, dtype=jnp.int32)
    pad = E2P - E2
    srcp = jnp.concatenate([edge_index[0], loop,
                            jnp.zeros((pad,), jnp.int32)])
    dstp = jnp.concatenate([edge_index[1], loop,
                            jnp.full((pad,), N, jnp.int32)])
    edges = (srcp << 14) | dstp

    hq = _emb(node_features, emb_W, emb_b)
    xlA, xlB, xrA, xrB = _mm2(hq, g0_Wl, g0_bl, g0_Wr, g0_br)
    hq = _sc_gat(xlA, xlB, xrA, xrB, edges, g0_att, g0_bias)
    xlA, xlB, xrA, xrB = _mm2(hq, g1_Wl, g1_bl, g1_Wr, g1_br)
    hq = _sc_gat(xlA, xlB, xrA, xrB, edges, g1_att, g1_bias)

    return _head(hq, batch_id, is_final, glob_W, glob_b, fc0_W, fc0_b,
                 fc1_W, fc1_b, fc2_W, fc2_b, ln_w, ln_b, q_W, q_b)
